# baseline (speedup 1.0000x reference)
import functools

import jax
import jax.numpy as jnp
from jax import lax
from jax.experimental import pallas as pl
from jax.experimental.pallas import tpu as pltpu
from jax.experimental.pallas import tpu_sc as plsc

_VOCAB = 1000
_VPAD = 1024
_NSUB = _VPAD // 128
_EMBED = 128
_BATCH = 1024
_NC = 2
_NS = 16
_NW = _NC * _NS
_BPW = _BATCH // _NW
_L = 16
_NFULL = _VOCAB // _L
_TAIL_START = _VOCAB - _L
_TAIL_KEEP = _NFULL * _L - _TAIL_START


def _mm_exp_body(v_ref, u_ref, em_ref):
    m = lax.dot_general(v_ref[...], u_ref[...],
                        dimension_numbers=(((1,), (1,)), ((), ())),
                        preferred_element_type=jnp.float32)
    em = jnp.exp(m)
    for s in range(_NSUB):
        em_ref[pl.ds(s * _VOCAB, _VOCAB), :] = em[:, s * 128:(s + 1) * 128]


def _mm_exp(v, u_pad):
    return pl.pallas_call(
        _mm_exp_body,
        out_shape=jax.ShapeDtypeStruct((_NSUB * _VOCAB, 128), jnp.float32),
    )(v, u_pad)


_sc_mesh = plsc.VectorSubcoreMesh(core_axis_name="c", subcore_axis_name="s")


@functools.partial(
    pl.kernel,
    mesh=_sc_mesh,
    compiler_params=pltpu.CompilerParams(
        use_tc_tiling_on_sc=False, needs_layout_passes=False,
        disable_bounds_checks=True, skip_device_barrier=True),
    out_type=(
        jax.ShapeDtypeStruct((_BATCH,), jnp.float32),
        jax.ShapeDtypeStruct((_BATCH,), jnp.float32),
    ),
    scratch_types=[
        pltpu.VMEM((_BPW,), jnp.int32),
        pltpu.VMEM((_BPW,), jnp.int32),
        pltpu.VMEM((_BPW * _NSUB // 2,), jnp.int32),
        pltpu.VMEM((_BPW * _NSUB // 2,), jnp.int32),
        pltpu.VMEM((_VOCAB, _BPW), jnp.int32),
        pltpu.VMEM((_BPW * _NSUB // 2, 128), jnp.float32),
        pltpu.VMEM((_BPW * _NSUB // 2, 128), jnp.float32),
        pltpu.VMEM((_BPW,), jnp.float32),
        pltpu.VMEM((_BPW,), jnp.float32),
        pltpu.SemaphoreType.DMA,
        pltpu.SemaphoreType.DMA,
        pltpu.SemaphoreType.DMA,
    ],
)
def _sc_gather(em_hbm, c_hbm, t_hbm, a_hbm, denom_hbm, escore_hbm,
               cidx, tidx, c8a, c8b, av, rows_a, rows_b, dstage, estage,
               sem_a, sem_b, sem_v):
    wid = lax.axis_index("s") * _NC + lax.axis_index("c")
    base = wid * _BPW
    pltpu.sync_copy(c_hbm.at[pl.ds(base, _BPW)], cidx)
    av_cp = pltpu.async_copy(a_hbm.at[:, pl.ds(base, _BPW)], av, sem_v)

    lanes = lax.iota(jnp.int32, _L)
    smul = (lanes & 7) * _VOCAB
    half = _BPW * _NSUB // 2
    for c in range(half // _L):
        rsel = (lanes >> 3) + 2 * c
        c8a[pl.ds(c * _L, _L)] = smul + plsc.load_gather(cidx, [rsel])
        c8b[pl.ds(c * _L, _L)] = smul + plsc.load_gather(cidx, [rsel + _L])
    cp_a = pltpu.async_copy(em_hbm.at[c8a], rows_a, sem_a)
    cp_b = pltpu.async_copy(em_hbm.at[c8b], rows_b, sem_b)
    pltpu.sync_copy(t_hbm.at[pl.ds(base, _BPW)], tidx)
    av_cp.wait()
    cp_a.wait()
    cp_b.wait()

    lanes8 = lanes * _NSUB
    _UNROLL = 4

    def _make_chunk(rows_ref, g):
        def _chunk(jo, acc):
            for jj in range(_UNROLL):
                col = av[jo * _UNROLL + jj, pl.ds(g * _L, _L)]
                acc = acc + plsc.load_gather(
                    rows_ref, [lanes8 + (col >> 7), col & 127])
            return acc
        return _chunk

    for g, rows_ref in enumerate((rows_a, rows_b)):
        dsums = lax.fori_loop(0, _VOCAB // _UNROLL, _make_chunk(rows_ref, g),
                              jnp.zeros((_L,), jnp.float32))
        dstage[pl.ds(g * _L, _L)] = dsums
        tcol = tidx[pl.ds(g * _L, _L)]
        estage[pl.ds(g * _L, _L)] = plsc.load_gather(
            rows_ref, [lanes * _NSUB + (tcol >> 7), tcol & 127])

    pltpu.sync_copy(dstage, denom_hbm.at[pl.ds(base, _BPW)])
    pltpu.sync_copy(estage, escore_hbm.at[pl.ds(base, _BPW)])


def _nll_body(d_ref, e_ref, o_ref):
    t = jnp.sum(jnp.log(d_ref[...])) - jnp.sum(jnp.log(e_ref[...]))
    o_ref[0, 0] = t * (1.0 / _BATCH)


def _nll(denom, escore):
    return pl.pallas_call(
        _nll_body,
        out_shape=jax.ShapeDtypeStruct((1, 1), jnp.float32),
        out_specs=pl.BlockSpec(memory_space=pltpu.SMEM),
    )(denom.reshape(8, 128), escore.reshape(8, 128))


def kernel(center_words, target_words, all_vocabs, embedding_v, embedding_u):
    c32 = center_words.reshape(-1).astype(jnp.int32)
    t32 = target_words.reshape(-1).astype(jnp.int32)
    a32 = all_vocabs.astype(jnp.int32).T
    u_pad = jnp.pad(embedding_u, ((0, _VPAD - _VOCAB), (0, 0)))
    em = _mm_exp(embedding_v, u_pad)
    denom, escore = _sc_gather(em, c32, t32, a32)
    return _nll(denom, escore)[0, 0]


# speedup vs baseline: 1.2056x; 1.2056x over previous
import functools

import jax
import jax.numpy as jnp
from jax import lax
from jax.experimental import pallas as pl
from jax.experimental.pallas import tpu as pltpu
from jax.experimental.pallas import tpu_sc as plsc

_VOCAB = 1000
_VPAD = 1024
_NSUB = _VPAD // 128
_EMBED = 128
_BATCH = 1024
_NC = 2
_NS = 16
_NW = _NC * _NS
_BPW = _BATCH // _NW
_L = 16
_NFULL = _VOCAB // _L
_TAIL_START = _VOCAB - _L
_TAIL_KEEP = _NFULL * _L - _TAIL_START


def _mm_exp_body(v_ref, u_ref, em_ref):
    m = lax.dot_general(v_ref[...], u_ref[...],
                        dimension_numbers=(((1,), (1,)), ((), ())),
                        preferred_element_type=jnp.float32)
    em = jnp.exp(m)
    for s in range(_NSUB - 1):
        em_ref[pl.ds(s * _VOCAB, _VOCAB), :] = em[:, s * 128:(s + 1) * 128]
    last = _VOCAB - 128 * (_NSUB - 1)
    em_ref[pl.ds((_NSUB - 1) * _VOCAB, _VOCAB), pl.ds(0, last)] = (
        em[:, 128 * (_NSUB - 1):])


def _mm_exp(v, u):
    return pl.pallas_call(
        _mm_exp_body,
        out_shape=jax.ShapeDtypeStruct((_NSUB * _VOCAB, 128), jnp.float32),
    )(v, u)


_sc_mesh = plsc.VectorSubcoreMesh(core_axis_name="c", subcore_axis_name="s")


@functools.partial(
    pl.kernel,
    mesh=_sc_mesh,
    compiler_params=pltpu.CompilerParams(
        use_tc_tiling_on_sc=False, needs_layout_passes=False,
        disable_bounds_checks=True, skip_device_barrier=True),
    out_type=(
        jax.ShapeDtypeStruct((_BATCH,), jnp.float32),
        jax.ShapeDtypeStruct((_BATCH,), jnp.float32),
    ),
    scratch_types=[
        pltpu.VMEM((_BPW,), jnp.int32),
        pltpu.VMEM((_BPW,), jnp.int32),
        pltpu.VMEM((_BPW * _NSUB // 2,), jnp.int32),
        pltpu.VMEM((_BPW * _NSUB // 2,), jnp.int32),
        pltpu.VMEM((_VOCAB // 8, 8, _BPW), jnp.int32),
        pltpu.VMEM((_BPW * _NSUB // 2, 128), jnp.float32),
        pltpu.VMEM((_BPW * _NSUB // 2, 128), jnp.float32),
        pltpu.VMEM((_BPW,), jnp.float32),
        pltpu.VMEM((_BPW,), jnp.float32),
        pltpu.SemaphoreType.DMA,
        pltpu.SemaphoreType.DMA,
        pltpu.SemaphoreType.DMA,
    ],
)
def _sc_gather(em_hbm, c_hbm, t_hbm, a_hbm, denom_hbm, escore_hbm,
               cidx, tidx, c8a, c8b, av, rows_a, rows_b, dstage, estage,
               sem_a, sem_b, sem_v):
    wid = lax.axis_index("s") * _NC + lax.axis_index("c")
    base = wid * _BPW
    pltpu.sync_copy(c_hbm.at[pl.ds(base, _BPW)], cidx)
    av_cp = pltpu.async_copy(
        a_hbm.at[:, wid >> 2, :, pl.ds((wid & 3) * _BPW, _BPW)], av, sem_v)

    lanes = lax.iota(jnp.int32, _L)
    smul = (lanes & 7) * _VOCAB
    half = _BPW * _NSUB // 2
    for c in range(half // _L):
        rsel = (lanes >> 3) + 2 * c
        c8a[pl.ds(c * _L, _L)] = smul + plsc.load_gather(cidx, [rsel])
        c8b[pl.ds(c * _L, _L)] = smul + plsc.load_gather(cidx, [rsel + _L])
    cp_a = pltpu.async_copy(em_hbm.at[c8a], rows_a, sem_a)
    cp_b = pltpu.async_copy(em_hbm.at[c8b], rows_b, sem_b)
    pltpu.sync_copy(t_hbm.at[pl.ds(base, _BPW)], tidx)
    av_cp.wait()
    cp_a.wait()
    cp_b.wait()

    lanes8 = lanes * _NSUB

    def _make_chunk(rows_ref, g):
        def _chunk(tj, acc):
            for jr in range(8):
                col = av[tj, jr, pl.ds(g * _L, _L)]
                acc = acc + plsc.load_gather(
                    rows_ref, [lanes8 + (col >> 7), col & 127])
            return acc
        return _chunk

    for g, rows_ref in enumerate((rows_a, rows_b)):
        dsums = lax.fori_loop(0, _VOCAB // 8, _make_chunk(rows_ref, g),
                              jnp.zeros((_L,), jnp.float32))
        dstage[pl.ds(g * _L, _L)] = dsums
        tcol = tidx[pl.ds(g * _L, _L)]
        estage[pl.ds(g * _L, _L)] = plsc.load_gather(
            rows_ref, [lanes * _NSUB + (tcol >> 7), tcol & 127])

    pltpu.sync_copy(dstage, denom_hbm.at[pl.ds(base, _BPW)])
    pltpu.sync_copy(estage, escore_hbm.at[pl.ds(base, _BPW)])


def _nll_body(d_ref, e_ref, o_ref):
    t = jnp.sum(jnp.log(d_ref[...])) - jnp.sum(jnp.log(e_ref[...]))
    o_ref[0, 0] = t * (1.0 / _BATCH)


def _nll(denom, escore):
    return pl.pallas_call(
        _nll_body,
        out_shape=jax.ShapeDtypeStruct((1, 1), jnp.float32),
        out_specs=pl.BlockSpec(memory_space=pltpu.SMEM),
    )(denom.reshape(8, 128), escore.reshape(8, 128))


def kernel(center_words, target_words, all_vocabs, embedding_v, embedding_u):
    c32 = center_words.reshape(-1).astype(jnp.int32)
    t32 = target_words.reshape(-1).astype(jnp.int32)
    a4 = (all_vocabs.astype(jnp.int32).T
          .reshape(_VOCAB // 8, 8, _BATCH // 128, 128)
          .transpose(0, 2, 1, 3))
    em = _mm_exp(embedding_v, embedding_u)
    denom, escore = _sc_gather(em, c32, t32, a4)
    return _nll(denom, escore)[0, 0]


# speedup vs baseline: 1.2245x; 1.0157x over previous
import functools

import jax
import jax.numpy as jnp
from jax import lax
from jax.experimental import pallas as pl
from jax.experimental.pallas import tpu as pltpu
from jax.experimental.pallas import tpu_sc as plsc

_VOCAB = 1000
_VPAD = 1024
_NSUB = _VPAD // 128
_EMBED = 128
_BATCH = 1024
_NC = 2
_NS = 16
_NW = _NC * _NS
_BPW = _BATCH // _NW
_L = 16
_NFULL = _VOCAB // _L
_TAIL_START = _VOCAB - _L
_TAIL_KEEP = _NFULL * _L - _TAIL_START


def _mm_exp_body(v_ref, u_ref, em_ref):
    m = lax.dot_general(v_ref[...], u_ref[...],
                        dimension_numbers=(((1,), (1,)), ((), ())),
                        preferred_element_type=jnp.float32)
    em = jnp.exp(m)
    for s in range(_NSUB - 1):
        em_ref[pl.ds(s * _VOCAB, _VOCAB), :] = em[:, s * 128:(s + 1) * 128]
    last = _VOCAB - 128 * (_NSUB - 1)
    em_ref[pl.ds((_NSUB - 1) * _VOCAB, _VOCAB), pl.ds(0, last)] = (
        em[:, 128 * (_NSUB - 1):])


def _mm_exp(v, u):
    return pl.pallas_call(
        _mm_exp_body,
        out_shape=jax.ShapeDtypeStruct((_NSUB * _VOCAB, 128), jnp.float32),
    )(v, u)


_sc_mesh = plsc.VectorSubcoreMesh(core_axis_name="c", subcore_axis_name="s")


@functools.partial(
    pl.kernel,
    mesh=_sc_mesh,
    compiler_params=pltpu.CompilerParams(
        use_tc_tiling_on_sc=False, needs_layout_passes=False,
        disable_bounds_checks=True, skip_device_barrier=True),
    out_type=(
        jax.ShapeDtypeStruct((_BATCH,), jnp.float32),
        jax.ShapeDtypeStruct((_BATCH,), jnp.float32),
    ),
    scratch_types=[
        pltpu.VMEM((_BPW,), jnp.int32),
        pltpu.VMEM((_BPW,), jnp.int32),
        pltpu.VMEM((_BPW * _NSUB // 2,), jnp.int32),
        pltpu.VMEM((_BPW * _NSUB // 2,), jnp.int32),
        pltpu.VMEM((_VOCAB // 8, 8, _L), jnp.int32),
        pltpu.VMEM((_VOCAB // 8, 8, _L), jnp.int32),
        pltpu.VMEM((_BPW * _NSUB // 2, 128), jnp.float32),
        pltpu.VMEM((_BPW * _NSUB // 2, 128), jnp.float32),
        pltpu.VMEM((_BPW,), jnp.float32),
        pltpu.VMEM((_BPW,), jnp.float32),
        pltpu.SemaphoreType.DMA,
        pltpu.SemaphoreType.DMA,
        pltpu.SemaphoreType.DMA,
        pltpu.SemaphoreType.DMA,
    ],
)
def _sc_gather(em_hbm, c_hbm, t_hbm, a_hbm, denom_hbm, escore_hbm,
               cidx, tidx, c8a, c8b, av_a, av_b, rows_a, rows_b,
               dstage, estage, sem_a, sem_b, sem_va, sem_vb):
    wid = lax.axis_index("s") * _NC + lax.axis_index("c")
    base = wid * _BPW
    pltpu.sync_copy(c_hbm.at[pl.ds(base, _BPW)], cidx)
    tb = wid >> 2
    bo = (wid & 3) * _BPW
    av_cp_a = pltpu.async_copy(
        a_hbm.at[:, tb, :, pl.ds(bo, _L)], av_a, sem_va)

    lanes = lax.iota(jnp.int32, _L)
    smul = (lanes & 7) * _VOCAB
    half = _BPW * _NSUB // 2
    for c in range(half // _L):
        rsel = (lanes >> 3) + 2 * c
        c8a[pl.ds(c * _L, _L)] = smul + plsc.load_gather(cidx, [rsel])
    cp_a = pltpu.async_copy(em_hbm.at[c8a], rows_a, sem_a)
    av_cp_b = pltpu.async_copy(
        a_hbm.at[:, tb, :, pl.ds(bo + _L, _L)], av_b, sem_vb)
    for c in range(half // _L):
        rsel = (lanes >> 3) + 2 * c
        c8b[pl.ds(c * _L, _L)] = smul + plsc.load_gather(cidx, [rsel + _L])
    cp_b = pltpu.async_copy(em_hbm.at[c8b], rows_b, sem_b)
    pltpu.sync_copy(t_hbm.at[pl.ds(base, _BPW)], tidx)

    lanes8 = lanes * _NSUB

    def _make_chunk(av_ref, rows_ref):
        def _chunk(tj, acc):
            for jr in range(8):
                col = av_ref[tj, jr, :]
                acc = acc + plsc.load_gather(
                    rows_ref, [lanes8 + (col >> 7), col & 127])
            return acc
        return _chunk

    for g, (av_ref, rows_ref, waits) in enumerate((
            (av_a, rows_a, (av_cp_a, cp_a)),
            (av_b, rows_b, (av_cp_b, cp_b)))):
        for w in waits:
            w.wait()
        dsums = lax.fori_loop(0, _VOCAB // 8, _make_chunk(av_ref, rows_ref),
                              jnp.zeros((_L,), jnp.float32))
        dstage[pl.ds(g * _L, _L)] = dsums
        tcol = tidx[pl.ds(g * _L, _L)]
        estage[pl.ds(g * _L, _L)] = plsc.load_gather(
            rows_ref, [lanes * _NSUB + (tcol >> 7), tcol & 127])

    pltpu.sync_copy(dstage, denom_hbm.at[pl.ds(base, _BPW)])
    pltpu.sync_copy(estage, escore_hbm.at[pl.ds(base, _BPW)])


def _nll_body(d_ref, e_ref, o_ref):
    t = jnp.sum(jnp.log(d_ref[...])) - jnp.sum(jnp.log(e_ref[...]))
    o_ref[0, 0] = t * (1.0 / _BATCH)


def _nll(denom, escore):
    return pl.pallas_call(
        _nll_body,
        out_shape=jax.ShapeDtypeStruct((1, 1), jnp.float32),
        out_specs=pl.BlockSpec(memory_space=pltpu.SMEM),
    )(denom.reshape(8, 128), escore.reshape(8, 128))


def kernel(center_words, target_words, all_vocabs, embedding_v, embedding_u):
    c32 = center_words.reshape(-1).astype(jnp.int32)
    t32 = target_words.reshape(-1).astype(jnp.int32)
    a4 = (all_vocabs.astype(jnp.int32).T
          .reshape(_VOCAB // 8, 8, _BATCH // 128, 128)
          .transpose(0, 2, 1, 3))
    em = _mm_exp(embedding_v, embedding_u)
    denom, escore = _sc_gather(em, c32, t32, a4)
    return _nll(denom, escore)[0, 0]
